# baseline (speedup 1.0000x reference)
import functools

import jax
import jax.numpy as jnp
from jax import lax
from jax.experimental import pallas as pl
from jax.experimental.pallas import tpu as pltpu
from jax.experimental.pallas import tpu_sc as plsc

B = 16384
L = 50
D = 32
NC, NS = 2, 16
NW = NC * NS
BPW = B // NW
LC = 2
RPC = LC * BPW
NCHUNK = L // LC
U = 8

_mesh = plsc.VectorSubcoreMesh(core_axis_name="c", subcore_axis_name="s")


@functools.partial(
    pl.kernel,
    out_type=jax.ShapeDtypeStruct((B, D), jnp.float32),
    mesh=_mesh,
    compiler_params=pltpu.CompilerParams(use_tc_tiling_on_sc=False),
    scratch_types=[
        pltpu.VMEM((L * BPW,), jnp.int32),
        pltpu.VMEM((RPC, D), jnp.float32),
        pltpu.VMEM((RPC, D), jnp.float32),
        pltpu.VMEM((BPW, D), jnp.float32),
        pltpu.SemaphoreType.DMA,
        pltpu.SemaphoreType.DMA,
        pltpu.SemaphoreType.DMA,
    ],
)
def _emb_bag(idx_hbm, w_hbm, out_hbm, idx_v, rows0, rows1, acc, si, s0, s1):
    wid = lax.axis_index("s") * NC + lax.axis_index("c")
    base = wid * BPW

    for l in range(L):
        pltpu.async_copy(
            idx_hbm.at[l, pl.ds(base, BPW)], idx_v.at[pl.ds(l * BPW, BPW)], si)
    for l in range(L):
        pltpu.make_async_copy(
            idx_hbm.at[l, pl.ds(base, BPW)], idx_v.at[pl.ds(l * BPW, BPW)], si
        ).wait()

    rows = (rows0, rows1)
    sems = (s0, s1)

    def start(g, b):
        pltpu.async_copy(
            w_hbm.at[idx_v.at[pl.ds(g * RPC, RPC)]], rows[b], sems[b])

    def wait(g, b):
        pltpu.make_async_copy(
            w_hbm.at[idx_v.at[pl.ds(g * RPC, RPC)]], rows[b], sems[b]
        ).wait()

    def accumulate(b, first):
        rbuf = rows[b]

        def body(c8, _):
            c0 = c8 * U
            for u in range(U):
                c = c0 + u
                v0 = rbuf[c, 0:16] + rbuf[c + BPW, 0:16]
                v1 = rbuf[c, 16:32] + rbuf[c + BPW, 16:32]
                if first:
                    acc[c, 0:16] = v0
                    acc[c, 16:32] = v1
                else:
                    plsc.addupdate(acc.at[c, 0:16], v0)
                    plsc.addupdate(acc.at[c, 16:32], v1)
            return 0

        lax.fori_loop(0, BPW // U, body, 0)

    start(0, 0)
    start(1, 1)
    wait(0, 0)
    accumulate(0, first=True)
    start(2, 0)
    wait(1, 1)
    accumulate(1, first=False)
    start(3, 1)

    def outer(g2, carry):
        for b in range(2):
            g = g2 * 2 + b
            wait(g, b)
            accumulate(b, first=False)

            @pl.when(g + 2 < NCHUNK)
            def _(b=b, g=g):
                start(g + 2, b)
        return carry

    lax.fori_loop(1, NCHUNK // 2, outer, 0)
    wait(NCHUNK - 1, 0)
    accumulate(0, first=False)

    pltpu.sync_copy(acc, out_hbm.at[pl.ds(base, BPW)])


def kernel(input, weight):
    return _emb_bag(input.astype(jnp.int32), weight)


# speedup vs baseline: 1.7023x; 1.7023x over previous
import functools

import jax
import jax.numpy as jnp
from jax import lax
from jax.experimental import pallas as pl
from jax.experimental.pallas import tpu as pltpu
from jax.experimental.pallas import tpu_sc as plsc

B = 16384
L = 50
D = 32
NC, NS = 2, 16
NW = NC * NS
BPW = B // NW
LC = 2
RPC = LC * BPW
NCHUNK = L // LC
U = 8

_VOCAB = 1000000
_TR = 8192
_TQ = _TR // 4
_TG = -(-_VOCAB // _TR)
_VPAD = _TG * _TR

_mesh = plsc.VectorSubcoreMesh(core_axis_name="c", subcore_axis_name="s")


@functools.partial(
    pl.kernel,
    out_type=jax.ShapeDtypeStruct((B, D), jnp.float32),
    mesh=_mesh,
    compiler_params=pltpu.CompilerParams(use_tc_tiling_on_sc=False),
    scratch_types=[
        pltpu.VMEM((L * BPW,), jnp.int32),
        pltpu.VMEM((RPC, D), jnp.float32),
        pltpu.VMEM((RPC, D), jnp.float32),
        pltpu.VMEM((BPW, D), jnp.float32),
        pltpu.SemaphoreType.DMA,
        pltpu.SemaphoreType.DMA,
        pltpu.SemaphoreType.DMA,
    ],
)
def _emb_bag(idx_hbm, w_hbm, out_hbm, idx_v, rows0, rows1, acc, si, s0, s1):
    wid = lax.axis_index("s") * NC + lax.axis_index("c")
    base = wid * BPW

    for l in range(L):
        pltpu.async_copy(
            idx_hbm.at[l, pl.ds(base, BPW)], idx_v.at[pl.ds(l * BPW, BPW)], si)
    for l in range(L):
        pltpu.make_async_copy(
            idx_hbm.at[l, pl.ds(base, BPW)], idx_v.at[pl.ds(l * BPW, BPW)], si
        ).wait()

    def xform(v8, _):
        off = v8 * (16 * U)
        for u in range(U):
            r = idx_v[pl.ds(off + u * 16, 16)]
            p = (r & ~jnp.int32(8191)) | ((r & jnp.int32(2047)) << 2) \
                | ((r >> 11) & jnp.int32(3))
            idx_v[pl.ds(off + u * 16, 16)] = p
        return 0

    lax.fori_loop(0, (L * BPW) // (16 * U), xform, 0)

    rows = (rows0, rows1)
    sems = (s0, s1)

    def start(g, b):
        pltpu.async_copy(
            w_hbm.at[idx_v.at[pl.ds(g * RPC, RPC)]], rows[b], sems[b])

    def wait(g, b):
        pltpu.make_async_copy(
            w_hbm.at[idx_v.at[pl.ds(g * RPC, RPC)]], rows[b], sems[b]
        ).wait()

    def accumulate(b, first):
        rbuf = rows[b]

        def body(c8, _):
            c0 = c8 * U
            for u in range(U):
                c = c0 + u
                v0 = rbuf[c, 0:16] + rbuf[c + BPW, 0:16]
                v1 = rbuf[c, 16:32] + rbuf[c + BPW, 16:32]
                if first:
                    acc[c, 0:16] = v0
                    acc[c, 16:32] = v1
                else:
                    plsc.addupdate(acc.at[c, 0:16], v0)
                    plsc.addupdate(acc.at[c, 16:32], v1)
            return 0

        lax.fori_loop(0, BPW // U, body, 0)

    start(0, 0)
    start(1, 1)
    wait(0, 0)
    accumulate(0, first=True)
    start(2, 0)
    wait(1, 1)
    accumulate(1, first=False)
    start(3, 1)

    def outer(g2, carry):
        for b in range(2):
            g = g2 * 2 + b
            wait(g, b)
            accumulate(b, first=False)

            @pl.when(g + 2 < NCHUNK)
            def _(b=b, g=g):
                start(g + 2, b)
        return carry

    lax.fori_loop(1, NCHUNK // 2, outer, 0)
    wait(NCHUNK - 1, 0)
    accumulate(0, first=False)

    pltpu.sync_copy(acc, out_hbm.at[pl.ds(base, BPW)])


def _t_body(x_ref, o_ref):
    x = x_ref[...]
    o_ref[...] = jnp.concatenate(
        [x[:, k * _TQ:(k + 1) * _TQ].T for k in range(4)], axis=1)


def _linearize_table(wt):
    return pl.pallas_call(
        _t_body,
        grid=(_TG,),
        in_specs=[pl.BlockSpec((32, _TR), lambda i: (0, i))],
        out_specs=pl.BlockSpec((_TQ, 128), lambda i: (i, 0)),
        out_shape=jax.ShapeDtypeStruct((_VPAD // 4, 128), jnp.float32),
    )(wt)


def kernel(input, weight):
    flat = _linearize_table(weight.T).reshape(_VPAD * D)
    table = flat.reshape(_VPAD, D)
    return _emb_bag(input.astype(jnp.int32), table)


# speedup vs baseline: 2.5361x; 1.4898x over previous
import functools

import jax
import jax.numpy as jnp
from jax import lax
from jax.experimental import pallas as pl
from jax.experimental.pallas import tpu as pltpu
from jax.experimental.pallas import tpu_sc as plsc

B = 16384
L = 50
D = 32
NC, NS = 2, 16
NW = NC * NS
BPW = B // NW
LC = 2
RPC = LC * BPW
NCHUNK = L // LC
U = 8

_VOCAB = 1000000
_TR = 8192
_TQ = _TR // 4
_TG = -(-_VOCAB // _TR)
_VPAD = _TG * _TR

_mesh = plsc.VectorSubcoreMesh(core_axis_name="c", subcore_axis_name="s")


@functools.partial(
    pl.kernel,
    out_type=jax.ShapeDtypeStruct((B, D), jnp.float32),
    mesh=_mesh,
    compiler_params=pltpu.CompilerParams(use_tc_tiling_on_sc=False),
    scratch_types=[
        pltpu.VMEM((L * BPW,), jnp.int32),
        pltpu.VMEM((RPC, D), jnp.float32),
        pltpu.VMEM((RPC, D), jnp.float32),
        pltpu.VMEM((BPW, D), jnp.float32),
        pltpu.SemaphoreType.DMA,
        pltpu.SemaphoreType.DMA,
        pltpu.SemaphoreType.DMA,
    ],
)
def _emb_bag(idx_hbm, w_hbm, out_hbm, idx_v, rows0, rows1, acc, si, s0, s1):
    wid = lax.axis_index("s") * NC + lax.axis_index("c")
    base = wid * BPW

    for l in range(L):
        pltpu.async_copy(
            idx_hbm.at[l, pl.ds(base, BPW)], idx_v.at[pl.ds(l * BPW, BPW)], si)
    for l in range(L):
        pltpu.make_async_copy(
            idx_hbm.at[l, pl.ds(base, BPW)], idx_v.at[pl.ds(l * BPW, BPW)], si
        ).wait()

    def xform(v8, _):
        off = v8 * (16 * U)
        for u in range(U):
            r = idx_v[pl.ds(off + u * 16, 16)]
            p = (r & ~jnp.int32(511)) | ((r & jnp.int32(127)) << 2) \
                | ((r >> 7) & jnp.int32(3))
            idx_v[pl.ds(off + u * 16, 16)] = p
        return 0

    lax.fori_loop(0, (L * BPW) // (16 * U), xform, 0)

    rows = (rows0, rows1)
    sems = (s0, s1)

    def start(g, b):
        pltpu.async_copy(
            w_hbm.at[idx_v.at[pl.ds(g * RPC, RPC)]], rows[b], sems[b])

    def wait(g, b):
        pltpu.make_async_copy(
            w_hbm.at[idx_v.at[pl.ds(g * RPC, RPC)]], rows[b], sems[b]
        ).wait()

    def accumulate(b, first):
        rbuf = rows[b]

        def body(c8, _):
            c0 = c8 * U
            for u in range(U):
                c = c0 + u
                v0 = rbuf[c, 0:16] + rbuf[c + BPW, 0:16]
                v1 = rbuf[c, 16:32] + rbuf[c + BPW, 16:32]
                if first:
                    acc[c, 0:16] = v0
                    acc[c, 16:32] = v1
                else:
                    plsc.addupdate(acc.at[c, 0:16], v0)
                    plsc.addupdate(acc.at[c, 16:32], v1)
            return 0

        lax.fori_loop(0, BPW // U, body, 0)

    start(0, 0)
    start(1, 1)
    wait(0, 0)
    accumulate(0, first=True)
    start(2, 0)
    wait(1, 1)
    accumulate(1, first=False)
    start(3, 1)

    def outer(g2, carry):
        for b in range(2):
            g = g2 * 2 + b
            wait(g, b)
            accumulate(b, first=False)

            @pl.when(g + 2 < NCHUNK)
            def _(b=b, g=g):
                start(g + 2, b)
        return carry

    lax.fori_loop(1, NCHUNK // 2, outer, 0)
    wait(NCHUNK - 1, 0)
    accumulate(0, first=False)

    pltpu.sync_copy(acc, out_hbm.at[pl.ds(base, BPW)])


def _t_body(x_ref, o_ref):
    x = x_ref[...]
    for t in range(_TR // 512):
        chunk = x[:, t * 512:(t + 1) * 512]
        z = jnp.concatenate(
            [chunk[:, q * 128:(q + 1) * 128] for q in range(4)], axis=0)
        o_ref[t * 128:(t + 1) * 128, :] = z.T


def _linearize_table(wt):
    return pl.pallas_call(
        _t_body,
        grid=(_TG,),
        in_specs=[pl.BlockSpec((32, _TR), lambda i: (0, i))],
        out_specs=pl.BlockSpec((_TQ, 128), lambda i: (i, 0)),
        out_shape=jax.ShapeDtypeStruct((_VPAD // 4, 128), jnp.float32),
    )(wt)


def kernel(input, weight):
    flat = _linearize_table(weight.T).reshape(_VPAD * D)
    table = flat.reshape(_VPAD, D)
    return _emb_bag(input.astype(jnp.int32), table)


# speedup vs baseline: 2.7054x; 1.0668x over previous
import functools

import jax
import jax.numpy as jnp
from jax import lax
from jax.experimental import pallas as pl
from jax.experimental.pallas import tpu as pltpu
from jax.experimental.pallas import tpu_sc as plsc

B = 16384
L = 50
D = 32
NC, NS = 2, 16
NW = NC * NS
BPW = B // NW
LC = 2
RPC = LC * BPW
NCHUNK = L // LC
U = 8

_VOCAB = 1000000
_TR = 8192
_TQ = _TR // 4
_TG = -(-_VOCAB // _TR)
_VPAD = _TG * _TR

_mesh = plsc.VectorSubcoreMesh(core_axis_name="c", subcore_axis_name="s")


@functools.partial(
    pl.kernel,
    out_type=jax.ShapeDtypeStruct((B, D), jnp.float32),
    mesh=_mesh,
    compiler_params=pltpu.CompilerParams(use_tc_tiling_on_sc=False),
    scratch_types=[
        pltpu.VMEM((L * BPW,), jnp.int32),
        pltpu.VMEM((BPW, D), jnp.float32),
        pltpu.SemaphoreType.DMA,
        pltpu.SemaphoreType.DMA,
    ],
)
def _emb_bag(idx_hbm, w_hbm, out_hbm, idx_v, acc, si, s0):
    wid = lax.axis_index("s") * NC + lax.axis_index("c")
    base = wid * BPW

    for l in range(L):
        pltpu.async_copy(
            idx_hbm.at[l, pl.ds(base, BPW)], idx_v.at[pl.ds(l * BPW, BPW)], si)
    for l in range(L):
        pltpu.make_async_copy(
            idx_hbm.at[l, pl.ds(base, BPW)], idx_v.at[pl.ds(l * BPW, BPW)], si
        ).wait()

    def xform(v8, _):
        off = v8 * (16 * U)
        for u in range(U):
            r = idx_v[pl.ds(off + u * 16, 16)]
            p = (r & ~jnp.int32(511)) | ((r & jnp.int32(127)) << 2) \
                | ((r >> 7) & jnp.int32(3))
            idx_v[pl.ds(off + u * 16, 16)] = p
        return 0

    lax.fori_loop(0, (L * BPW) // (16 * U), xform, 0)

    zero = jnp.zeros((16,), jnp.float32)

    def zbody(z8, _):
        z0 = z8 * U
        for u in range(U):
            acc[z0 + u, 0:16] = zero
            acc[z0 + u, 16:32] = zero
        return 0

    lax.fori_loop(0, BPW // U, zbody, 0)

    for l in range(L):
        pltpu.async_copy(
            w_hbm.at[idx_v.at[pl.ds(l * BPW, BPW)]], acc, s0, add=True)
    for l in range(L):
        pltpu.make_async_copy(
            w_hbm.at[idx_v.at[pl.ds(l * BPW, BPW)]], acc, s0).wait()

    pltpu.sync_copy(acc, out_hbm.at[pl.ds(base, BPW)])


def _t_body(x_ref, o_ref):
    x = x_ref[...]
    for t in range(_TR // 512):
        chunk = x[:, t * 512:(t + 1) * 512]
        z = jnp.concatenate(
            [chunk[:, q * 128:(q + 1) * 128] for q in range(4)], axis=0)
        o_ref[t * 128:(t + 1) * 128, :] = z.T


def _linearize_table(wt):
    return pl.pallas_call(
        _t_body,
        grid=(_TG,),
        in_specs=[pl.BlockSpec((32, _TR), lambda i: (0, i))],
        out_specs=pl.BlockSpec((_TQ, 128), lambda i: (i, 0)),
        out_shape=jax.ShapeDtypeStruct((_VPAD // 4, 128), jnp.float32),
    )(wt)


def kernel(input, weight):
    flat = _linearize_table(weight.T).reshape(_VPAD * D)
    table = flat.reshape(_VPAD, D)
    return _emb_bag(input.astype(jnp.int32), table)


# speedup vs baseline: 3.3152x; 1.2254x over previous
import functools

import jax
import jax.numpy as jnp
from jax import lax
from jax.experimental import pallas as pl
from jax.experimental.pallas import tpu as pltpu
from jax.experimental.pallas import tpu_sc as plsc

B = 16384
L = 50
D = 32
NC, NS = 2, 16
NW = NC * NS
BPW = B // NW
LC = 2
RPC = LC * BPW
NCHUNK = L // LC
U = 8

_VOCAB = 1000000
_TR = 16384
_TQ = _TR // 4
_TG = -(-_VOCAB // _TR)
_VPAD = _TG * _TR

_mesh = plsc.VectorSubcoreMesh(core_axis_name="c", subcore_axis_name="s")


@functools.partial(
    pl.kernel,
    out_type=jax.ShapeDtypeStruct((B, D), jnp.float32),
    mesh=_mesh,
    compiler_params=pltpu.CompilerParams(use_tc_tiling_on_sc=False),
    scratch_types=[
        pltpu.VMEM((L * BPW,), jnp.int32),
        pltpu.VMEM((BPW, D), jnp.float32),
        pltpu.SemaphoreType.DMA,
        pltpu.SemaphoreType.DMA,
    ],
)
def _emb_bag(idx_hbm, w_hbm, out_hbm, idx_v, acc, si, s0):
    wid = lax.axis_index("s") * NC + lax.axis_index("c")
    base = wid * BPW

    for l in range(L):
        pltpu.async_copy(
            idx_hbm.at[l, pl.ds(base, BPW)], idx_v.at[pl.ds(l * BPW, BPW)], si)
    for l in range(L):
        pltpu.make_async_copy(
            idx_hbm.at[l, pl.ds(base, BPW)], idx_v.at[pl.ds(l * BPW, BPW)], si
        ).wait()

    def xform(v8, _):
        off = v8 * (16 * U)
        for u in range(U):
            r = idx_v[pl.ds(off + u * 16, 16)]
            p = (r & ~jnp.int32(511)) | ((r & jnp.int32(127)) << 2) \
                | ((r >> 7) & jnp.int32(3))
            idx_v[pl.ds(off + u * 16, 16)] = p
        return 0

    lax.fori_loop(0, (L * BPW) // (16 * U), xform, 0)

    zero = jnp.zeros((16,), jnp.float32)

    def zbody(z8, _):
        z0 = z8 * U
        for u in range(U):
            acc[z0 + u, 0:16] = zero
            acc[z0 + u, 16:32] = zero
        return 0

    lax.fori_loop(0, BPW // U, zbody, 0)

    for l in range(L):
        pltpu.async_copy(
            w_hbm.at[idx_v.at[pl.ds(l * BPW, BPW)]], acc, s0, add=True)
    for l in range(L):
        pltpu.make_async_copy(
            w_hbm.at[idx_v.at[pl.ds(l * BPW, BPW)]], acc, s0).wait()

    pltpu.sync_copy(acc, out_hbm.at[pl.ds(base, BPW)])


def _t_body(x_ref, o_ref):
    x = x_ref[...]
    for t in range(_TR // 512):
        chunk = x[:, t * 512:(t + 1) * 512]
        z = jnp.concatenate(
            [chunk[:, q * 128:(q + 1) * 128] for q in range(4)], axis=0)
        o_ref[t * 128:(t + 1) * 128, :] = z.T


def _linearize_table(wt):
    return pl.pallas_call(
        _t_body,
        grid=(_TG,),
        in_specs=[pl.BlockSpec((32, _TR), lambda i: (0, i))],
        out_specs=pl.BlockSpec((_TQ, 128), lambda i: (i, 0)),
        out_shape=jax.ShapeDtypeStruct((_VPAD // 4, 128), jnp.float32),
    )(wt)


def kernel(input, weight):
    flat = _linearize_table(weight.T).reshape(_VPAD * D)
    table = flat.reshape(_VPAD, D)
    return _emb_bag(input.astype(jnp.int32), table)


# speedup vs baseline: 3.6030x; 1.0868x over previous
import functools

import jax
import jax.numpy as jnp
from jax import lax
from jax.experimental import pallas as pl
from jax.experimental.pallas import tpu as pltpu
from jax.experimental.pallas import tpu_sc as plsc

B = 16384
L = 50
D = 32
NC, NS = 2, 16
NW = NC * NS
BPW = B // NW
LC = 2
RPC = LC * BPW
NCHUNK = L // LC
U = 8

_VOCAB = 1000000
_TR = 32768
_TQ = _TR // 4
_TG = -(-_VOCAB // _TR)
_VPAD = _TG * _TR

_mesh = plsc.VectorSubcoreMesh(core_axis_name="c", subcore_axis_name="s")


@functools.partial(
    pl.kernel,
    out_type=jax.ShapeDtypeStruct((B, D), jnp.float32),
    mesh=_mesh,
    compiler_params=pltpu.CompilerParams(use_tc_tiling_on_sc=False),
    scratch_types=[
        pltpu.VMEM((L * BPW,), jnp.int32),
        pltpu.VMEM((BPW, D), jnp.float32),
        pltpu.SemaphoreType.DMA,
        pltpu.SemaphoreType.DMA,
    ],
)
def _emb_bag(idx_hbm, w_hbm, out_hbm, idx_v, acc, si, s0):
    wid = lax.axis_index("s") * NC + lax.axis_index("c")
    base = wid * BPW

    for l in range(L):
        pltpu.async_copy(
            idx_hbm.at[l, pl.ds(base, BPW)], idx_v.at[pl.ds(l * BPW, BPW)], si)
    for l in range(L):
        pltpu.make_async_copy(
            idx_hbm.at[l, pl.ds(base, BPW)], idx_v.at[pl.ds(l * BPW, BPW)], si
        ).wait()

    def xform(v8, _):
        off = v8 * (16 * U)
        for u in range(U):
            r = idx_v[pl.ds(off + u * 16, 16)]
            p = (r & ~jnp.int32(511)) | ((r & jnp.int32(127)) << 2) \
                | ((r >> 7) & jnp.int32(3))
            idx_v[pl.ds(off + u * 16, 16)] = p
        return 0

    lax.fori_loop(0, (L * BPW) // (16 * U), xform, 0)

    zero = jnp.zeros((16,), jnp.float32)

    def zbody(z8, _):
        z0 = z8 * U
        for u in range(U):
            acc[z0 + u, 0:16] = zero
            acc[z0 + u, 16:32] = zero
        return 0

    lax.fori_loop(0, BPW // U, zbody, 0)

    for l in range(L):
        pltpu.async_copy(
            w_hbm.at[idx_v.at[pl.ds(l * BPW, BPW)]], acc, s0, add=True)
    for l in range(L):
        pltpu.make_async_copy(
            w_hbm.at[idx_v.at[pl.ds(l * BPW, BPW)]], acc, s0).wait()

    pltpu.sync_copy(acc, out_hbm.at[pl.ds(base, BPW)])


def _t_body(x_ref, o_ref):
    x = x_ref[...]
    for t in range(_TR // 512):
        chunk = x[:, t * 512:(t + 1) * 512]
        z = jnp.concatenate(
            [chunk[:, q * 128:(q + 1) * 128] for q in range(4)], axis=0)
        o_ref[t * 128:(t + 1) * 128, :] = z.T


def _linearize_table(wt):
    return pl.pallas_call(
        _t_body,
        grid=(_TG,),
        in_specs=[pl.BlockSpec((32, _TR), lambda i: (0, i))],
        out_specs=pl.BlockSpec((_TQ, 128), lambda i: (i, 0)),
        out_shape=jax.ShapeDtypeStruct((_VPAD // 4, 128), jnp.float32),
    )(wt)


def kernel(input, weight):
    flat = _linearize_table(weight.T).reshape(_VPAD * D)
    table = flat.reshape(_VPAD, D)
    return _emb_bag(input.astype(jnp.int32), table)


# speedup vs baseline: 3.6479x; 1.0125x over previous
import functools

import jax
import jax.numpy as jnp
from jax import lax
from jax.experimental import pallas as pl
from jax.experimental.pallas import tpu as pltpu
from jax.experimental.pallas import tpu_sc as plsc

B = 16384
L = 50
D = 32
NC, NS = 2, 16
NW = NC * NS
BPW = B // NW
LC = 2
RPC = LC * BPW
NCHUNK = L // LC
U = 8

_VOCAB = 1000000
_TR = 65536
_TQ = _TR // 4
_TG = -(-_VOCAB // _TR)
_VPAD = _TG * _TR

_mesh = plsc.VectorSubcoreMesh(core_axis_name="c", subcore_axis_name="s")


@functools.partial(
    pl.kernel,
    out_type=jax.ShapeDtypeStruct((B, D), jnp.float32),
    mesh=_mesh,
    compiler_params=pltpu.CompilerParams(use_tc_tiling_on_sc=False),
    scratch_types=[
        pltpu.VMEM((L * BPW,), jnp.int32),
        pltpu.VMEM((BPW, D), jnp.float32),
        pltpu.SemaphoreType.DMA,
        pltpu.SemaphoreType.DMA,
    ],
)
def _emb_bag(idx_hbm, w_hbm, out_hbm, idx_v, acc, si, s0):
    wid = lax.axis_index("s") * NC + lax.axis_index("c")
    base = wid * BPW

    for l in range(L):
        pltpu.async_copy(
            idx_hbm.at[l, pl.ds(base, BPW)], idx_v.at[pl.ds(l * BPW, BPW)], si)
    for l in range(L):
        pltpu.make_async_copy(
            idx_hbm.at[l, pl.ds(base, BPW)], idx_v.at[pl.ds(l * BPW, BPW)], si
        ).wait()

    def xform(v8, _):
        off = v8 * (16 * U)
        for u in range(U):
            r = idx_v[pl.ds(off + u * 16, 16)]
            p = (r & ~jnp.int32(511)) | ((r & jnp.int32(127)) << 2) \
                | ((r >> 7) & jnp.int32(3))
            idx_v[pl.ds(off + u * 16, 16)] = p
        return 0

    lax.fori_loop(0, (L * BPW) // (16 * U), xform, 0)

    zero = jnp.zeros((16,), jnp.float32)

    def zbody(z8, _):
        z0 = z8 * U
        for u in range(U):
            acc[z0 + u, 0:16] = zero
            acc[z0 + u, 16:32] = zero
        return 0

    lax.fori_loop(0, BPW // U, zbody, 0)

    for l in range(L):
        pltpu.async_copy(
            w_hbm.at[idx_v.at[pl.ds(l * BPW, BPW)]], acc, s0, add=True)
    for l in range(L):
        pltpu.make_async_copy(
            w_hbm.at[idx_v.at[pl.ds(l * BPW, BPW)]], acc, s0).wait()

    pltpu.sync_copy(acc, out_hbm.at[pl.ds(base, BPW)])


def _t_body(x_ref, o_ref):
    x = x_ref[...]
    for t in range(_TR // 512):
        chunk = x[:, t * 512:(t + 1) * 512]
        z = jnp.concatenate(
            [chunk[:, q * 128:(q + 1) * 128] for q in range(4)], axis=0)
        o_ref[t * 128:(t + 1) * 128, :] = z.T


def _linearize_table(wt):
    return pl.pallas_call(
        _t_body,
        grid=(_TG,),
        in_specs=[pl.BlockSpec((32, _TR), lambda i: (0, i))],
        out_specs=pl.BlockSpec((_TQ, 128), lambda i: (i, 0)),
        out_shape=jax.ShapeDtypeStruct((_VPAD // 4, 128), jnp.float32),
    )(wt)


def kernel(input, weight):
    flat = _linearize_table(weight.T).reshape(_VPAD * D)
    table = flat.reshape(_VPAD, D)
    return _emb_bag(input.astype(jnp.int32), table)


# speedup vs baseline: 3.6834x; 1.0097x over previous
import functools

import jax
import jax.numpy as jnp
from jax import lax
from jax.experimental import pallas as pl
from jax.experimental.pallas import tpu as pltpu
from jax.experimental.pallas import tpu_sc as plsc

B = 16384
L = 50
D = 32
NC, NS = 2, 16
NW = NC * NS
BPW = B // NW
LC = 2
RPC = LC * BPW
NCHUNK = L // LC
U = 8

_VOCAB = 1000000
_TR = 65536
_TQ = _TR // 4
_TG = -(-_VOCAB // _TR)
_VPAD = _TG * _TR

_mesh = plsc.VectorSubcoreMesh(core_axis_name="c", subcore_axis_name="s")


@functools.partial(
    pl.kernel,
    out_type=jax.ShapeDtypeStruct((B, D), jnp.float32),
    mesh=_mesh,
    compiler_params=pltpu.CompilerParams(use_tc_tiling_on_sc=False),
    scratch_types=[
        pltpu.VMEM((L * BPW,), jnp.int32),
        pltpu.VMEM((BPW, D), jnp.float32),
        pltpu.SemaphoreType.DMA,
        pltpu.SemaphoreType.DMA,
    ],
)
def _emb_bag(idx_hbm, w_hbm, out_hbm, idx_v, acc, si, s0):
    wid = lax.axis_index("s") * NC + lax.axis_index("c")
    base = wid * BPW

    for l in range(L):
        pltpu.async_copy(
            idx_hbm.at[l, pl.ds(base, BPW)], idx_v.at[pl.ds(l * BPW, BPW)], si)
    for l in range(L):
        pltpu.make_async_copy(
            idx_hbm.at[l, pl.ds(base, BPW)], idx_v.at[pl.ds(l * BPW, BPW)], si
        ).wait()

    zero = jnp.zeros((16,), jnp.float32)

    def zbody(z8, _):
        z0 = z8 * U
        for u in range(U):
            acc[z0 + u, 0:16] = zero
            acc[z0 + u, 16:32] = zero
        return 0

    lax.fori_loop(0, BPW // U, zbody, 0)

    for l in range(L):
        pltpu.async_copy(
            w_hbm.at[idx_v.at[pl.ds(l * BPW, BPW)]], acc, s0, add=True)
    for l in range(L):
        pltpu.make_async_copy(
            w_hbm.at[idx_v.at[pl.ds(l * BPW, BPW)]], acc, s0).wait()

    pltpu.sync_copy(acc, out_hbm.at[pl.ds(base, BPW)])


def _t_body(x_ref, o_ref):
    x = x_ref[...]
    for t in range(_TR // 512):
        chunk = x[:, t * 512:(t + 1) * 512]
        z = jnp.concatenate(
            [chunk[:, q * 128:(q + 1) * 128] for q in range(4)], axis=0)
        o_ref[t * 128:(t + 1) * 128, :] = z.T


def _linearize_table(wt):
    return pl.pallas_call(
        _t_body,
        grid=(_TG,),
        in_specs=[pl.BlockSpec((32, _TR), lambda i: (0, i))],
        out_specs=pl.BlockSpec((_TQ, 128), lambda i: (i, 0)),
        out_shape=jax.ShapeDtypeStruct((_VPAD // 4, 128), jnp.float32),
    )(wt)


def kernel(input, weight):
    flat = _linearize_table(weight.T).reshape(_VPAD * D)
    table = flat.reshape(_VPAD, D)
    r = input.astype(jnp.int32)
    pidx = (r & ~jnp.int32(511)) | ((r & jnp.int32(127)) << 2) \
        | ((r >> 7) & jnp.int32(3))
    return _emb_bag(pidx, table)


# speedup vs baseline: 3.7231x; 1.0108x over previous
import functools

import jax
import jax.numpy as jnp
from jax import lax
from jax.experimental import pallas as pl
from jax.experimental.pallas import tpu as pltpu
from jax.experimental.pallas import tpu_sc as plsc

B = 16384
L = 50
D = 32
NC, NS = 2, 16
NW = NC * NS
BPW = B // NW
LC = 2
RPC = LC * BPW
NCHUNK = L // LC
U = 8

_VOCAB = 1000000
_TR = 65536
_TQ = _TR // 4
_TG = -(-_VOCAB // _TR)
_VPAD = _TG * _TR

_mesh = plsc.VectorSubcoreMesh(core_axis_name="c", subcore_axis_name="s")


@functools.partial(
    pl.kernel,
    out_type=jax.ShapeDtypeStruct((B, D), jnp.float32),
    mesh=_mesh,
    compiler_params=pltpu.CompilerParams(use_tc_tiling_on_sc=False),
    scratch_types=[
        pltpu.VMEM((L * BPW,), jnp.int32),
        pltpu.VMEM((BPW, D), jnp.float32),
        pltpu.SemaphoreType.DMA,
        pltpu.SemaphoreType.DMA,
    ],
)
def _emb_bag(idx_hbm, w_hbm, out_hbm, idx_v, acc, si, s0):
    wid = lax.axis_index("s") * NC + lax.axis_index("c")
    base = wid * BPW

    for l in range(L):
        pltpu.async_copy(
            idx_hbm.at[l, pl.ds(base, BPW)], idx_v.at[pl.ds(l * BPW, BPW)], si)

    zero = jnp.zeros((16,), jnp.float32)

    def zbody(z8, _):
        z0 = z8 * U
        for u in range(U):
            acc[z0 + u, 0:16] = zero
            acc[z0 + u, 16:32] = zero
        return 0

    lax.fori_loop(0, BPW // U, zbody, 0)

    for l in range(L):
        pltpu.make_async_copy(
            idx_hbm.at[l, pl.ds(base, BPW)], idx_v.at[pl.ds(l * BPW, BPW)], si
        ).wait()
        pltpu.async_copy(
            w_hbm.at[idx_v.at[pl.ds(l * BPW, BPW)]], acc, s0, add=True)
    for l in range(L):
        pltpu.make_async_copy(
            w_hbm.at[idx_v.at[pl.ds(l * BPW, BPW)]], acc, s0).wait()

    pltpu.sync_copy(acc, out_hbm.at[pl.ds(base, BPW)])


def _t_body(x_ref, o_ref):
    x = x_ref[...]
    for t in range(_TR // 512):
        chunk = x[:, t * 512:(t + 1) * 512]
        z = jnp.concatenate(
            [chunk[:, q * 128:(q + 1) * 128] for q in range(4)], axis=0)
        o_ref[t * 128:(t + 1) * 128, :] = z.T


def _linearize_table(wt):
    return pl.pallas_call(
        _t_body,
        grid=(_TG,),
        in_specs=[pl.BlockSpec((32, _TR), lambda i: (0, i))],
        out_specs=pl.BlockSpec((_TQ, 128), lambda i: (i, 0)),
        out_shape=jax.ShapeDtypeStruct((_VPAD // 4, 128), jnp.float32),
    )(wt)


def kernel(input, weight):
    flat = _linearize_table(weight.T).reshape(_VPAD * D)
    table = flat.reshape(_VPAD, D)
    r = input.astype(jnp.int32)
    pidx = (r & ~jnp.int32(511)) | ((r & jnp.int32(127)) << 2) \
        | ((r >> 7) & jnp.int32(3))
    return _emb_bag(pidx, table)
